# baseline (speedup 1.0000x reference)
import functools

import jax
import jax.numpy as jnp
from jax import lax
from jax.experimental import pallas as pl
from jax.experimental.pallas import tpu as pltpu
from jax.experimental.pallas import tpu_sc as plsc

N1 = 4096
N2 = 4096
E = 16384
D = 128
DW = 16
NC = 2
NS = 16
NW = NC * NS
EPW = E // NW
C = 64
NCH = EPW // C
L = 16
RPT = N1 // NS
EPS = 1e-8


_NBLK = 4
_NR = N1 // _NBLK


def _norms_body(x1_ref, x2_ref, r1_ref, r2_ref):
    x1 = x1_ref[...]
    r1_ref[...] = jnp.sqrt(jnp.sum(x1 * x1, axis=1)).reshape(_NR // D, D)
    x2 = x2_ref[...]
    r2_ref[...] = jnp.sqrt(jnp.sum(x2 * x2, axis=1)).reshape(_NR // D, D)


_norms = pl.pallas_call(
    _norms_body,
    grid=(_NBLK,),
    in_specs=[
        pl.BlockSpec((_NR, D), lambda i: (i, 0)),
        pl.BlockSpec((_NR, D), lambda i: (i, 0)),
    ],
    out_specs=[
        pl.BlockSpec((_NR // D, D), lambda i: (i, 0)),
        pl.BlockSpec((_NR // D, D), lambda i: (i, 0)),
    ],
    out_shape=[
        jax.ShapeDtypeStruct((N1 // D, D), jnp.float32),
        jax.ShapeDtypeStruct((N2 // D, D), jnp.float32),
    ],
)


def _gates_body(xn_ref, wg_ref, gates_ref):
    g = lax.dot_general(xn_ref[...], wg_ref[...],
                        (((1,), (1,)), ((), ())),
                        preferred_element_type=jnp.float32)
    gates_ref[...] = jax.nn.sigmoid(g)


_gates = pl.pallas_call(
    _gates_body,
    out_shape=jax.ShapeDtypeStruct((N1, D), jnp.float32),
)


def _sc_body(x1_hbm, x2_hbm, idx_hbm, r1_hbm, r2_hbm,
             outf_hbm, outd_hbm,
             src2d, dst2d, r1t, r2t, x1b, x2b, stf, std, pbt, sbuf, qbuf,
             accf, accd, g1s0, g1s1, g1s2, g2s0, g2s1, g2s2, ssemf, ssemd):
    g1s = (g1s0, g1s1, g1s2)
    g2s = (g2s0, g2s1, g2s2)
    cid = lax.axis_index("c")
    sid = lax.axis_index("s")
    wid = cid * NS + sid
    lane = lax.iota(jnp.int32, L)
    zv = jnp.zeros((L,), jnp.float32)

    pltpu.sync_copy(idx_hbm.at[0].at[pl.ds(wid * NCH, NCH)], src2d)
    pltpu.sync_copy(idx_hbm.at[1].at[pl.ds(wid * NCH, NCH)], dst2d)
    pltpu.sync_copy(r1_hbm, r1t)
    pltpu.sync_copy(r2_hbm, r2t)

    def zbody(e):
        for b in range(2):
            for u in range(D // L):
                stf[b, e, pl.ds(u * L, L)] = zv
            std[b, e, pl.ds(0, L)] = zv

    plsc.parallel_loop(0, C, 1, unroll=2)(zbody)
    for t in range(RPT // C):
        pltpu.sync_copy(stf.at[0], accf.at[pl.ds(sid * RPT + t * C, C)])
        pltpu.sync_copy(std.at[0], accd.at[pl.ds(sid * RPT + t * C, C)])

    plsc.subcore_barrier()

    NB = 3

    def _start_gathers(cj):
        gb = cj % NB
        return (
            pltpu.async_copy(x1_hbm.at[src2d.at[cj]], x1b.at[gb], g1s[gb]),
            pltpu.async_copy(x2_hbm.at[dst2d.at[cj]], x2b.at[gb], g2s[gb]),
        )

    gcp = {0: _start_gathers(0), 1: _start_gathers(1)}
    scp = {}

    for ci in range(NCH):
        b = ci % NB
        sb = ci % 2
        cp1, cp2 = gcp[ci]
        cp1.wait()
        cp2.wait()
        if ci + 2 < NCH:
            gcp[ci + 2] = _start_gathers(ci + 2)
        if ci >= 2:
            scp[ci - 2][0].wait()
            scp[ci - 2][1].wait()

        def p1(e, b=b):
            a0 = x1b[b, e, pl.ds(0, L)] * x2b[b, e, pl.ds(0, L)]
            a1 = x1b[b, e, pl.ds(L, L)] * x2b[b, e, pl.ds(L, L)]
            for u in range(2, D // L, 2):
                a0 = a0 + x1b[b, e, pl.ds(u * L, L)] * x2b[b, e, pl.ds(u * L, L)]
                a1 = a1 + x1b[b, e, pl.ds((u + 1) * L, L)] * x2b[b, e, pl.ds((u + 1) * L, L)]
            col = jnp.zeros((L,), jnp.int32) + e
            plsc.store_scatter(pbt, [lane, col], a0 + a1)

        plsc.parallel_loop(0, C, 1, unroll=2)(p1)

        for g in range(C // L):
            ev = src2d[ci, pl.ds(g * L, L)]
            dv = dst2d[ci, pl.ds(g * L, L)]
            r1v = plsc.load_gather(r1t, [ev >> 7, ev & 127])
            r2v = plsc.load_gather(r2t, [dv >> 7, dv & 127])
            den = jnp.maximum(r1v * r2v, EPS)
            row = lane + g * L
            t = [pbt[j, pl.ds(g * L, L)] for j in range(L)]
            while len(t) > 1:
                t = [t[i] + t[i + 1] for i in range(0, len(t), 2)]
            s = jnp.exp(t[0] / den)
            sbuf[pl.ds(g * L, L)] = s
            plsc.store_scatter(std.at[sb],
                               [row, jnp.zeros((L,), jnp.int32)], s)

        def p3(e, b=b, sb=sb):
            sv = plsc.load_gather(sbuf, [jnp.zeros((L,), jnp.int32) + e])
            for u in range(D // L):
                stf[sb, e, pl.ds(u * L, L)] = x2b[b, e, pl.ds(u * L, L)] * sv

        plsc.parallel_loop(0, C, 1, unroll=2)(p3)

        scp[ci] = (
            pltpu.async_copy(stf.at[sb], accf.at[src2d.at[ci]], ssemf,
                             add=True),
            pltpu.async_copy(std.at[sb], accd.at[src2d.at[ci]], ssemd,
                             add=True),
        )

    for ci in (NCH - 2, NCH - 1):
        scp[ci][0].wait()
        scp[ci][1].wait()
    plsc.subcore_barrier()
    pltpu.sync_copy(accf.at[pl.ds(sid * RPT, RPT)],
                    outf_hbm.at[cid].at[pl.ds(sid * RPT, RPT)])
    zi = jnp.zeros((L,), jnp.int32)
    for t in range(RPT // C):
        pltpu.sync_copy(accd.at[pl.ds(sid * RPT + t * C, C)], std.at[0])
        for g in range(C // L):
            dv = plsc.load_gather(std.at[0], [lane + g * L, zi])
            off = t * C + g * L
            qbuf[off // D, pl.ds(off % D, L)] = dv
    pltpu.sync_copy(qbuf, outd_hbm.at[cid].at[pl.ds(sid * (RPT // D), RPT // D)])


_sc_edges = functools.partial(
    pl.kernel,
    out_type=[
        jax.ShapeDtypeStruct((NC, N1, D), jnp.float32),
        jax.ShapeDtypeStruct((NC, N1 // D, D), jnp.float32),
    ],
    mesh=plsc.VectorSubcoreMesh(core_axis_name="c", subcore_axis_name="s"),
    compiler_params=pltpu.CompilerParams(use_tc_tiling_on_sc=False,
                                         needs_layout_passes=False,
                                         disable_bounds_checks=True),
    scratch_types=[
        pltpu.VMEM((E // C // NW, C), jnp.int32),
        pltpu.VMEM((E // C // NW, C), jnp.int32),
        pltpu.VMEM((N1 // D, D), jnp.float32),
        pltpu.VMEM((N2 // D, D), jnp.float32),
        pltpu.VMEM((3, C, D), jnp.float32),
        pltpu.VMEM((3, C, D), jnp.float32),
        pltpu.VMEM((2, C, D), jnp.float32),
        pltpu.VMEM((2, C, DW), jnp.float32),
        pltpu.VMEM((L, C), jnp.float32),
        pltpu.VMEM((C,), jnp.float32),
        pltpu.VMEM((N1 // NS // D, D), jnp.float32),
        pltpu.VMEM_SHARED((N1, D), jnp.float32),
        pltpu.VMEM_SHARED((N1, DW), jnp.float32),
        pltpu.SemaphoreType.DMA,
        pltpu.SemaphoreType.DMA,
        pltpu.SemaphoreType.DMA,
        pltpu.SemaphoreType.DMA,
        pltpu.SemaphoreType.DMA,
        pltpu.SemaphoreType.DMA,
        pltpu.SemaphoreType.DMA,
        pltpu.SemaphoreType.DMA,
    ],
)(_sc_body)


def _combine_body(pf_ref, pd_ref, gates_ref, out_ref):
    gates = gates_ref[...]
    num = pf_ref[0] + pf_ref[1]
    d2 = pd_ref[0] + pd_ref[1]
    e1 = jnp.repeat(d2, D, axis=0)
    m = (lax.broadcasted_iota(jnp.int32, (_NR, D), 0) % D ==
         lax.broadcasted_iota(jnp.int32, (_NR, D), 1))
    den = jnp.sum(jnp.where(m, e1, 0.0), axis=1, keepdims=True)
    safe = jnp.where(den > 0, den, 1.0)
    out_ref[...] = jnp.where(den > 0, gates * (num / safe), 0.0)


_combine = pl.pallas_call(
    _combine_body,
    grid=(_NBLK,),
    in_specs=[
        pl.BlockSpec((NC, _NR, D), lambda i: (0, i, 0)),
        pl.BlockSpec((NC, _NR // D, D), lambda i: (0, i, 0)),
        pl.BlockSpec((_NR, D), lambda i: (i, 0)),
    ],
    out_specs=pl.BlockSpec((_NR, D), lambda i: (i, 0)),
    out_shape=jax.ShapeDtypeStruct((N1, D), jnp.float32),
)


def kernel(X_h_1, X_h_2, X_n_1, cross_indices, W_gate):
    idx3 = cross_indices.astype(jnp.int32).reshape(2, E // C, C)
    r1, r2 = _norms(X_h_1, X_h_2)
    pf, pd = _sc_edges(X_h_1, X_h_2, idx3, r1, r2)
    gates = _gates(X_n_1, W_gate)
    return _combine(pf, pd, gates)


# speedup vs baseline: 1.0115x; 1.0115x over previous
import functools

import jax
import jax.numpy as jnp
from jax import lax
from jax.experimental import pallas as pl
from jax.experimental.pallas import tpu as pltpu
from jax.experimental.pallas import tpu_sc as plsc

N1 = 4096
N2 = 4096
E = 16384
D = 128
DW = 16
NC = 2
NS = 16
NW = NC * NS
EPW = E // NW
C = 64
NCH = EPW // C
L = 16
RPT = N1 // NS
EPS = 1e-8


def _norms_body(x1_ref, x2_ref, r1_ref, r2_ref):
    x1 = x1_ref[...]
    r1_ref[...] = jnp.sqrt(jnp.sum(x1 * x1, axis=1)).reshape(N1 // D, D)
    x2 = x2_ref[...]
    r2_ref[...] = jnp.sqrt(jnp.sum(x2 * x2, axis=1)).reshape(N2 // D, D)


_norms = pl.pallas_call(
    _norms_body,
    out_shape=[
        jax.ShapeDtypeStruct((N1 // D, D), jnp.float32),
        jax.ShapeDtypeStruct((N2 // D, D), jnp.float32),
    ],
)


def _gates_body(xn_ref, wg_ref, gates_ref):
    g = lax.dot_general(xn_ref[...], wg_ref[...],
                        (((1,), (1,)), ((), ())),
                        preferred_element_type=jnp.float32)
    gates_ref[...] = jax.nn.sigmoid(g)


_gates = pl.pallas_call(
    _gates_body,
    out_shape=jax.ShapeDtypeStruct((N1, D), jnp.float32),
)


def _sc_body(x1_hbm, x2_hbm, idx_hbm, r1_hbm, r2_hbm,
             outf_hbm, outd_hbm,
             src2d, dst2d, r1t, r2t, x1b, x2b, stf, std, pbt, sbuf, qbuf,
             accf, accd, g1s0, g1s1, g1s2, g2s0, g2s1, g2s2, ssemf, ssemd):
    g1s = (g1s0, g1s1, g1s2)
    g2s = (g2s0, g2s1, g2s2)
    cid = lax.axis_index("c")
    sid = lax.axis_index("s")
    wid = cid * NS + sid
    lane = lax.iota(jnp.int32, L)
    zv = jnp.zeros((L,), jnp.float32)

    pltpu.sync_copy(idx_hbm.at[0].at[pl.ds(wid * NCH, NCH)], src2d)
    pltpu.sync_copy(idx_hbm.at[1].at[pl.ds(wid * NCH, NCH)], dst2d)
    pltpu.sync_copy(r1_hbm, r1t)
    pltpu.sync_copy(r2_hbm, r2t)

    def zbody(e):
        for b in range(2):
            for u in range(D // L):
                stf[b, e, pl.ds(u * L, L)] = zv
            std[b, e, pl.ds(0, L)] = zv

    plsc.parallel_loop(0, C, 1, unroll=2)(zbody)
    for t in range(RPT // C):
        pltpu.sync_copy(stf.at[0], accf.at[pl.ds(sid * RPT + t * C, C)])
        pltpu.sync_copy(std.at[0], accd.at[pl.ds(sid * RPT + t * C, C)])

    plsc.subcore_barrier()

    NB = 3

    def _start_gathers(cj):
        gb = cj % NB
        return (
            pltpu.async_copy(x1_hbm.at[src2d.at[cj]], x1b.at[gb], g1s[gb]),
            pltpu.async_copy(x2_hbm.at[dst2d.at[cj]], x2b.at[gb], g2s[gb]),
        )

    gcp = {0: _start_gathers(0), 1: _start_gathers(1)}
    scp = {}

    for ci in range(NCH):
        b = ci % NB
        sb = ci % 2
        cp1, cp2 = gcp[ci]
        cp1.wait()
        cp2.wait()
        if ci + 2 < NCH:
            gcp[ci + 2] = _start_gathers(ci + 2)
        if ci >= 2:
            scp[ci - 2][0].wait()
            scp[ci - 2][1].wait()

        def p1(e, b=b):
            a0 = x1b[b, e, pl.ds(0, L)] * x2b[b, e, pl.ds(0, L)]
            a1 = x1b[b, e, pl.ds(L, L)] * x2b[b, e, pl.ds(L, L)]
            for u in range(2, D // L, 2):
                a0 = a0 + x1b[b, e, pl.ds(u * L, L)] * x2b[b, e, pl.ds(u * L, L)]
                a1 = a1 + x1b[b, e, pl.ds((u + 1) * L, L)] * x2b[b, e, pl.ds((u + 1) * L, L)]
            col = jnp.zeros((L,), jnp.int32) + e
            plsc.store_scatter(pbt, [lane, col], a0 + a1)

        plsc.parallel_loop(0, C, 1, unroll=2)(p1)

        for g in range(C // L):
            ev = src2d[ci, pl.ds(g * L, L)]
            dv = dst2d[ci, pl.ds(g * L, L)]
            r1v = plsc.load_gather(r1t, [ev >> 7, ev & 127])
            r2v = plsc.load_gather(r2t, [dv >> 7, dv & 127])
            den = jnp.maximum(r1v * r2v, EPS)
            row = lane + g * L
            t = [pbt[j, pl.ds(g * L, L)] for j in range(L)]
            while len(t) > 1:
                t = [t[i] + t[i + 1] for i in range(0, len(t), 2)]
            s = jnp.exp(t[0] / den)
            sbuf[pl.ds(g * L, L)] = s
            plsc.store_scatter(std.at[sb],
                               [row, jnp.zeros((L,), jnp.int32)], s)

        def p3(e, b=b, sb=sb):
            sv = plsc.load_gather(sbuf, [jnp.zeros((L,), jnp.int32) + e])
            for u in range(D // L):
                stf[sb, e, pl.ds(u * L, L)] = x2b[b, e, pl.ds(u * L, L)] * sv

        plsc.parallel_loop(0, C, 1, unroll=2)(p3)

        scp[ci] = (
            pltpu.async_copy(stf.at[sb], accf.at[src2d.at[ci]], ssemf,
                             add=True),
            pltpu.async_copy(std.at[sb], accd.at[src2d.at[ci]], ssemd,
                             add=True),
        )

    for ci in (NCH - 2, NCH - 1):
        scp[ci][0].wait()
        scp[ci][1].wait()
    plsc.subcore_barrier()
    pltpu.sync_copy(accf.at[pl.ds(sid * RPT, RPT)],
                    outf_hbm.at[cid].at[pl.ds(sid * RPT, RPT)])
    zi = jnp.zeros((L,), jnp.int32)
    for t in range(RPT // C):
        pltpu.sync_copy(accd.at[pl.ds(sid * RPT + t * C, C)], std.at[0])
        for g in range(C // L):
            dv = plsc.load_gather(std.at[0], [lane + g * L, zi])
            off = t * C + g * L
            qbuf[off // D, pl.ds(off % D, L)] = dv
    pltpu.sync_copy(qbuf, outd_hbm.at[cid].at[pl.ds(sid * (RPT // D), RPT // D)])


_sc_edges = functools.partial(
    pl.kernel,
    out_type=[
        jax.ShapeDtypeStruct((NC, N1, D), jnp.float32),
        jax.ShapeDtypeStruct((NC, N1 // D, D), jnp.float32),
    ],
    mesh=plsc.VectorSubcoreMesh(core_axis_name="c", subcore_axis_name="s"),
    compiler_params=pltpu.CompilerParams(use_tc_tiling_on_sc=False,
                                         needs_layout_passes=False,
                                         disable_bounds_checks=True),
    scratch_types=[
        pltpu.VMEM((E // C // NW, C), jnp.int32),
        pltpu.VMEM((E // C // NW, C), jnp.int32),
        pltpu.VMEM((N1 // D, D), jnp.float32),
        pltpu.VMEM((N2 // D, D), jnp.float32),
        pltpu.VMEM((3, C, D), jnp.float32),
        pltpu.VMEM((3, C, D), jnp.float32),
        pltpu.VMEM((2, C, D), jnp.float32),
        pltpu.VMEM((2, C, DW), jnp.float32),
        pltpu.VMEM((L, C), jnp.float32),
        pltpu.VMEM((C,), jnp.float32),
        pltpu.VMEM((N1 // NS // D, D), jnp.float32),
        pltpu.VMEM_SHARED((N1, D), jnp.float32),
        pltpu.VMEM_SHARED((N1, DW), jnp.float32),
        pltpu.SemaphoreType.DMA,
        pltpu.SemaphoreType.DMA,
        pltpu.SemaphoreType.DMA,
        pltpu.SemaphoreType.DMA,
        pltpu.SemaphoreType.DMA,
        pltpu.SemaphoreType.DMA,
        pltpu.SemaphoreType.DMA,
        pltpu.SemaphoreType.DMA,
    ],
)(_sc_body)


def _combine_body(pf_ref, pd_ref, gates_ref, out_ref):
    gates = gates_ref[...]
    num = pf_ref[0] + pf_ref[1]
    d2 = pd_ref[0] + pd_ref[1]
    e1 = jnp.repeat(d2, D, axis=0)
    m = (lax.broadcasted_iota(jnp.int32, (N1, D), 0) % D ==
         lax.broadcasted_iota(jnp.int32, (N1, D), 1))
    den = jnp.sum(jnp.where(m, e1, 0.0), axis=1, keepdims=True)
    safe = jnp.where(den > 0, den, 1.0)
    out_ref[...] = jnp.where(den > 0, gates * (num / safe), 0.0)


_combine = pl.pallas_call(
    _combine_body,
    out_shape=jax.ShapeDtypeStruct((N1, D), jnp.float32),
)


def kernel(X_h_1, X_h_2, X_n_1, cross_indices, W_gate):
    idx3 = cross_indices.astype(jnp.int32).reshape(2, E // C, C)
    r1, r2 = _norms(X_h_1, X_h_2)
    pf, pd = _sc_edges(X_h_1, X_h_2, idx3, r1, r2)
    gates = _gates(X_n_1, W_gate)
    return _combine(pf, pd, gates)
